# baseline (speedup 1.0000x reference)
import functools

import jax
import jax.numpy as jnp
from jax import lax
from jax.experimental import pallas as pl
from jax.experimental.pallas import tpu as pltpu
from jax.experimental.pallas import tpu_sc as plsc

WINDOW = 50
PAD_WINDOW = 56
CHUNK = 2 * PAD_WINDOW
NBUF = 4


def _build_lookup(num_workers: int, n_chunks: int, vocab: int, dim: int):
    mesh = plsc.VectorSubcoreMesh(core_axis_name="c", subcore_axis_name="s")
    num_cores = 2
    batches_per_worker = 2 * n_chunks
    total_batches = num_workers * batches_per_worker
    n_outer = n_chunks // NBUF

    @functools.partial(
        pl.kernel,
        mesh=mesh,
        out_type=jax.ShapeDtypeStruct((total_batches, PAD_WINDOW, dim),
                                      jnp.float32),
        scratch_types=(
            [pltpu.VMEM((n_chunks, CHUNK), jnp.int32)]
            + [pltpu.VMEM((CHUNK, dim), jnp.float32) for _ in range(NBUF)]
            + [pltpu.SemaphoreType.DMA for _ in range(2 * NBUF)]
        ),
    )
    def lookup(idx_hbm, table_hbm, out_hbm, idx_v, *rest):
        bufs = rest[:NBUF]
        gsems = rest[NBUF:2 * NBUF]
        wsems = rest[2 * NBUF:]
        wid = lax.axis_index("s") * num_cores + lax.axis_index("c")
        batch0 = wid * batches_per_worker
        pltpu.sync_copy(idx_hbm.at[wid], idx_v)

        def gather(j, b):
            return pltpu.make_async_copy(
                table_hbm.at[idx_v.at[j]], bufs[b], gsems[b])

        def writebacks(j, b):
            bb = batch0 + 2 * j
            return [
                pltpu.make_async_copy(
                    bufs[b].at[pl.ds(0, PAD_WINDOW)],
                    out_hbm.at[bb], wsems[b]),
                pltpu.make_async_copy(
                    bufs[b].at[pl.ds(PAD_WINDOW, PAD_WINDOW)],
                    out_hbm.at[bb + 1], wsems[b]),
            ]

        for b in range(NBUF):
            gather(b, b).start()

        def body(i, carry):
            j0 = i * NBUF
            for b in range(NBUF):
                j = j0 + b
                gather(j, b).wait()
                for wb in writebacks(j, b):
                    wb.start()
                for wb in writebacks(j, b):
                    wb.wait()
                gather(j + NBUF, b).start()
            return carry

        lax.fori_loop(0, n_outer - 1, body, 0)

        j0 = (n_outer - 1) * NBUF
        for b in range(NBUF):
            j = j0 + b
            gather(j, b).wait()
            for wb in writebacks(j, b):
                wb.start()
        for b in range(NBUF):
            for wb in writebacks(j0 + b, b):
                wb.wait()

    return lookup


def kernel(inputs, initial_state, embedding_table):
    batch, window = inputs.shape
    vocab, dim = embedding_table.shape
    num_workers = 32
    assert window == WINDOW
    padded = jnp.pad(inputs, ((0, 0), (0, PAD_WINDOW - WINDOW)))
    total = batch * PAD_WINDOW
    assert total % (num_workers * CHUNK) == 0
    n_chunks = total // (num_workers * CHUNK)
    assert n_chunks % NBUF == 0
    idx = padded.reshape(num_workers, n_chunks, CHUNK)
    out = _build_lookup(num_workers, n_chunks, vocab, dim)(idx, embedding_table)
    return out[:, :WINDOW, :]


# speedup vs baseline: 1.0012x; 1.0012x over previous
import functools

import jax
import jax.numpy as jnp
from jax import lax
from jax.experimental import pallas as pl
from jax.experimental.pallas import tpu as pltpu
from jax.experimental.pallas import tpu_sc as plsc

WINDOW = 50
PAD_WINDOW = 56
CHUNK = 2 * PAD_WINDOW
NBUF = 4


def _build_lookup(num_workers: int, n_chunks: int, vocab: int, dim: int):
    mesh = plsc.VectorSubcoreMesh(core_axis_name="c", subcore_axis_name="s")
    num_cores = 2
    batches_per_worker = 2 * n_chunks
    total_batches = num_workers * batches_per_worker
    n_outer = n_chunks // NBUF

    @functools.partial(
        pl.kernel,
        mesh=mesh,
        out_type=jax.ShapeDtypeStruct((total_batches * PAD_WINDOW, dim),
                                      jnp.float32),
        scratch_types=(
            [pltpu.VMEM((n_chunks, CHUNK), jnp.int32)]
            + [pltpu.VMEM((CHUNK, dim), jnp.float32) for _ in range(NBUF)]
            + [pltpu.SemaphoreType.DMA for _ in range(2 * NBUF)]
        ),
    )
    def lookup(idx_hbm, table_hbm, out_hbm, idx_v, *rest):
        bufs = rest[:NBUF]
        gsems = rest[NBUF:2 * NBUF]
        wsems = rest[2 * NBUF:]
        wid = lax.axis_index("s") * num_cores + lax.axis_index("c")
        batch0 = wid * batches_per_worker
        pltpu.sync_copy(idx_hbm.at[wid], idx_v)

        def gather(j, b):
            return pltpu.make_async_copy(
                table_hbm.at[idx_v.at[j]], bufs[b], gsems[b])

        def writebacks(j, b):
            row0 = (batch0 + 2 * j) * PAD_WINDOW
            return [
                pltpu.make_async_copy(
                    bufs[b], out_hbm.at[pl.ds(row0, CHUNK)], wsems[b]),
            ]

        for b in range(NBUF):
            gather(b, b).start()

        def body(i, carry):
            j0 = i * NBUF
            for b in range(NBUF):
                j = j0 + b
                gather(j, b).wait()
                for wb in writebacks(j, b):
                    wb.start()
                for wb in writebacks(j, b):
                    wb.wait()
                gather(j + NBUF, b).start()
            return carry

        lax.fori_loop(0, n_outer - 1, body, 0)

        j0 = (n_outer - 1) * NBUF
        for b in range(NBUF):
            j = j0 + b
            gather(j, b).wait()
            for wb in writebacks(j, b):
                wb.start()
        for b in range(NBUF):
            for wb in writebacks(j0 + b, b):
                wb.wait()

    return lookup


def kernel(inputs, initial_state, embedding_table):
    batch, window = inputs.shape
    vocab, dim = embedding_table.shape
    num_workers = 32
    assert window == WINDOW
    padded = jnp.pad(inputs, ((0, 0), (0, PAD_WINDOW - WINDOW)))
    total = batch * PAD_WINDOW
    assert total % (num_workers * CHUNK) == 0
    n_chunks = total // (num_workers * CHUNK)
    assert n_chunks % NBUF == 0
    idx = padded.reshape(num_workers, n_chunks, CHUNK)
    out = _build_lookup(num_workers, n_chunks, vocab, dim)(idx, embedding_table)
    return out.reshape(batch, PAD_WINDOW, dim)[:, :WINDOW, :]


# speedup vs baseline: 7.8291x; 7.8194x over previous
import functools

import jax
import jax.numpy as jnp
from jax import lax
from jax.experimental import pallas as pl
from jax.experimental.pallas import tpu as pltpu
from jax.experimental.pallas import tpu_sc as plsc

WINDOW = 50
PAD_WINDOW = 56
WPC = 8
NBUF = 2


def _build_lookup(num_workers: int, n_chunks: int, vocab: int, dim: int):
    mesh = plsc.VectorSubcoreMesh(core_axis_name="c", subcore_axis_name="s")
    num_cores = 2
    batches_per_worker = n_chunks * WPC
    total_batches = num_workers * batches_per_worker
    idx_per_worker = batches_per_worker * PAD_WINDOW
    n_outer = n_chunks // NBUF

    @functools.partial(
        pl.kernel,
        mesh=mesh,
        out_type=jax.ShapeDtypeStruct((total_batches, WINDOW, dim),
                                      jnp.float32),
        scratch_types=(
            [pltpu.VMEM((idx_per_worker,), jnp.int32)]
            + [pltpu.VMEM((WPC, WINDOW, dim), jnp.float32)
               for _ in range(NBUF)]
            + [pltpu.SemaphoreType.DMA for _ in range(2 * NBUF)]
        ),
    )
    def lookup(idx_hbm, table_hbm, out_hbm, idx_v, *rest):
        bufs = rest[:NBUF]
        gsems = rest[NBUF:2 * NBUF]
        wsems = rest[2 * NBUF:]
        wid = lax.axis_index("s") * num_cores + lax.axis_index("c")
        batch0 = wid * batches_per_worker
        pltpu.sync_copy(idx_hbm.at[pl.ds(wid * idx_per_worker, idx_per_worker)],
                        idx_v)

        def gathers(j, b):
            return [
                pltpu.make_async_copy(
                    table_hbm.at[idx_v.at[pl.ds((j * WPC + k) * PAD_WINDOW,
                                                WINDOW)]],
                    bufs[b].at[k], gsems[b])
                for k in range(WPC)
            ]

        def writeback(j, b):
            return pltpu.make_async_copy(
                bufs[b], out_hbm.at[pl.ds(batch0 + j * WPC, WPC)], wsems[b])

        for b in range(NBUF):
            for g in gathers(b, b):
                g.start()

        def body(i, carry):
            j0 = i * NBUF
            for b in range(NBUF):
                j = j0 + b
                for g in gathers(j, b):
                    g.wait()
                writeback(j, b).start()
                writeback(j, b).wait()
                for g in gathers(j + NBUF, b):
                    g.start()
            return carry

        lax.fori_loop(0, n_outer - 1, body, 0)

        j0 = (n_outer - 1) * NBUF
        for b in range(NBUF):
            j = j0 + b
            for g in gathers(j, b):
                g.wait()
            writeback(j, b).start()
        for b in range(NBUF):
            writeback(j0 + b, b).wait()

    return lookup


def kernel(inputs, initial_state, embedding_table):
    batch, window = inputs.shape
    vocab, dim = embedding_table.shape
    num_workers = 32
    assert window == WINDOW
    idx = jnp.pad(inputs, ((0, 0), (0, PAD_WINDOW - WINDOW))).reshape(-1)
    assert batch % (num_workers * WPC) == 0
    n_chunks = batch // (num_workers * WPC)
    assert n_chunks % NBUF == 0
    return _build_lookup(num_workers, n_chunks, vocab, dim)(
        idx, embedding_table)


# speedup vs baseline: 13.8955x; 1.7749x over previous
import functools

import jax
import jax.numpy as jnp
from jax import lax
from jax.experimental import pallas as pl
from jax.experimental.pallas import tpu as pltpu
from jax.experimental.pallas import tpu_sc as plsc

CHUNK = 64
NBUF = 10


def _build_lookup(num_workers: int, n_chunks: int, vocab: int, dim: int):
    mesh = plsc.VectorSubcoreMesh(core_axis_name="c", subcore_axis_name="s")
    num_cores = 2
    rows_per_worker = n_chunks * CHUNK
    n_outer = n_chunks // NBUF

    @functools.partial(
        pl.kernel,
        mesh=mesh,
        out_type=jax.ShapeDtypeStruct((num_workers * rows_per_worker, dim),
                                      jnp.float32),
        scratch_types=(
            [pltpu.VMEM((n_chunks, CHUNK), jnp.int32)]
            + [pltpu.VMEM((CHUNK, dim), jnp.float32) for _ in range(NBUF)]
            + [pltpu.SemaphoreType.DMA for _ in range(2 * NBUF)]
        ),
    )
    def lookup(idx_hbm, table_hbm, out_hbm, idx_v, *rest):
        bufs = rest[:NBUF]
        gsems = rest[NBUF:2 * NBUF]
        wsems = rest[2 * NBUF:]
        wid = lax.axis_index("s") * num_cores + lax.axis_index("c")
        base = wid * rows_per_worker
        pltpu.sync_copy(idx_hbm.at[wid], idx_v)

        def gather(j, b):
            return pltpu.make_async_copy(
                table_hbm.at[idx_v.at[j]], bufs[b], gsems[b])

        def writeback(j, b):
            return pltpu.make_async_copy(
                bufs[b], out_hbm.at[pl.ds(base + j * CHUNK, CHUNK)], wsems[b])

        for b in range(NBUF):
            gather(b, b).start()

        def body(i, carry):
            j0 = i * NBUF
            for b in range(NBUF):
                j = j0 + b
                gather(j, b).wait()
                writeback(j, b).start()
                writeback(j, b).wait()
                gather(j + NBUF, b).start()
            return carry

        lax.fori_loop(0, n_outer - 1, body, 0)

        j0 = (n_outer - 1) * NBUF
        for b in range(NBUF):
            j = j0 + b
            gather(j, b).wait()
            writeback(j, b).start()
        for b in range(NBUF):
            writeback(j0 + b, b).wait()

    return lookup


def kernel(inputs, initial_state, embedding_table):
    batch, window = inputs.shape
    vocab, dim = embedding_table.shape
    total = batch * window
    num_workers = 32
    assert total % (num_workers * CHUNK) == 0
    n_chunks = total // (num_workers * CHUNK)
    assert n_chunks % NBUF == 0
    idx = inputs.T.reshape(num_workers, n_chunks, CHUNK)
    out = _build_lookup(num_workers, n_chunks, vocab, dim)(idx, embedding_table)
    return out.reshape(window, batch, dim).transpose(1, 0, 2)


# speedup vs baseline: 13.9837x; 1.0063x over previous
import functools

import jax
import jax.numpy as jnp
from jax import lax
from jax.experimental import pallas as pl
from jax.experimental.pallas import tpu as pltpu
from jax.experimental.pallas import tpu_sc as plsc

CHUNK = 64
NBUF = 10


def _build_lookup(num_workers: int, n_chunks: int, vocab: int, dim: int):
    mesh = plsc.VectorSubcoreMesh(core_axis_name="c", subcore_axis_name="s")
    num_cores = 2
    rows_per_worker = n_chunks * CHUNK
    n_outer = n_chunks // NBUF

    @functools.partial(
        pl.kernel,
        mesh=mesh,
        out_type=jax.ShapeDtypeStruct((num_workers * rows_per_worker, dim),
                                      jnp.float32),
        scratch_types=(
            [pltpu.VMEM((rows_per_worker,), jnp.int32)]
            + [pltpu.VMEM((CHUNK, dim), jnp.float32) for _ in range(NBUF)]
            + [pltpu.SemaphoreType.DMA for _ in range(2 * NBUF)]
        ),
    )
    def lookup(idx_hbm, table_hbm, out_hbm, idx_v, *rest):
        bufs = rest[:NBUF]
        gsems = rest[NBUF:2 * NBUF]
        wsems = rest[2 * NBUF:]
        wid = lax.axis_index("s") * num_cores + lax.axis_index("c")
        base = wid * rows_per_worker
        pltpu.sync_copy(idx_hbm.at[pl.ds(base, rows_per_worker)], idx_v)

        def gather(j, b):
            return pltpu.make_async_copy(
                table_hbm.at[idx_v.at[pl.ds(j * CHUNK, CHUNK)]],
                bufs[b], gsems[b])

        def writeback(j, b):
            return pltpu.make_async_copy(
                bufs[b], out_hbm.at[pl.ds(base + j * CHUNK, CHUNK)], wsems[b])

        for b in range(NBUF):
            gather(b, b).start()

        def body(i, carry):
            j0 = i * NBUF
            for b in range(NBUF):
                j = j0 + b
                gather(j, b).wait()
                writeback(j, b).start()
                writeback(j, b).wait()
                gather(j + NBUF, b).start()
            return carry

        lax.fori_loop(0, n_outer - 1, body, 0)

        j0 = (n_outer - 1) * NBUF
        for b in range(NBUF):
            j = j0 + b
            gather(j, b).wait()
            writeback(j, b).start()
        for b in range(NBUF):
            writeback(j0 + b, b).wait()

    return lookup


def kernel(inputs, initial_state, embedding_table):
    batch, window = inputs.shape
    vocab, dim = embedding_table.shape
    total = batch * window
    num_workers = 32
    assert total % (num_workers * CHUNK) == 0
    n_chunks = total // (num_workers * CHUNK)
    assert n_chunks % NBUF == 0
    idx = inputs.T.reshape(-1)
    out = _build_lookup(num_workers, n_chunks, vocab, dim)(idx, embedding_table)
    return out.reshape(window, batch, dim).transpose(1, 0, 2)
